# baseline (speedup 1.0000x reference)
import functools

import jax
import jax.numpy as jnp
from jax import lax
from jax.experimental import pallas as pl
from jax.experimental.pallas import tpu as pltpu
from jax.experimental.pallas import tpu_sc as plsc

HIDDEN = 64
GAMMA_VAL = 12.0
BATCH_N = 16384
LANES = 16

NUM_CORES = 2
NUM_SUBCORES = 16
NW = NUM_CORES * NUM_SUBCORES
BW = BATCH_N // NW
CHUNK = 128
NCHUNK = BW // CHUNK


def _score_body(samp, ent, rel, out,
                samp_v, hidx_v, ridx_v, tidx_v, h_v, r_v, t_v, out_v, sem):
    wid = lax.axis_index("s") * NUM_CORES + lax.axis_index("c")
    base = wid * BW

    pltpu.sync_copy(samp.at[wid], samp_v)

    iota = lax.iota(jnp.int32, LANES)
    stride3 = iota * 3
    per_chunk = CHUNK // LANES
    for g in range(BW // LANES):
        c, k = divmod(g, per_chunk)
        sl = pl.ds(k * LANES, LANES)
        hidx_v[c, sl] = plsc.load_gather(samp_v, [stride3 + (g * LANES * 3)])
        ridx_v[c, sl] = plsc.load_gather(samp_v, [stride3 + (g * LANES * 3 + 1)])
        tidx_v[c, sl] = plsc.load_gather(samp_v, [stride3 + (g * LANES * 3 + 2)])

    copies = []
    for c in range(NCHUNK):
        dst = pl.ds(c * CHUNK, CHUNK)
        copies.append(pltpu.async_copy(ent.at[hidx_v.at[c]], h_v.at[dst], sem))
        copies.append(pltpu.async_copy(rel.at[ridx_v.at[c]], r_v.at[dst], sem))
        copies.append(pltpu.async_copy(ent.at[tidx_v.at[c]], t_v.at[dst], sem))
    for cp in copies:
        cp.wait()

    def body(g, carry):
        row = iota + g * LANES
        acc = jnp.zeros((LANES,), jnp.float32)
        for d in range(HIDDEN):
            col = jnp.full((LANES,), d, jnp.int32)
            h = plsc.load_gather(h_v, [row, col])
            r = plsc.load_gather(r_v, [row, col])
            t = plsc.load_gather(t_v, [row, col])
            acc = acc + jnp.abs(h + r - t)
        out_v[pl.ds(g * LANES, LANES)] = GAMMA_VAL - acc
        return carry

    lax.fori_loop(0, BW // LANES, body, 0)
    pltpu.sync_copy(out_v, out.at[pl.ds(base, BW)])


_sc_call = pl.kernel(
    _score_body,
    out_type=jax.ShapeDtypeStruct((BATCH_N,), jnp.float32),
    mesh=plsc.VectorSubcoreMesh(core_axis_name="c", subcore_axis_name="s"),
    scratch_types=[
        pltpu.VMEM((BW * 3,), jnp.int32),
        pltpu.VMEM((NCHUNK, CHUNK), jnp.int32),
        pltpu.VMEM((NCHUNK, CHUNK), jnp.int32),
        pltpu.VMEM((NCHUNK, CHUNK), jnp.int32),
        pltpu.VMEM((BW, HIDDEN), jnp.float32),
        pltpu.VMEM((BW, HIDDEN), jnp.float32),
        pltpu.VMEM((BW, HIDDEN), jnp.float32),
        pltpu.VMEM((BW,), jnp.float32),
        pltpu.SemaphoreType.DMA,
    ],
    compiler_params=pltpu.CompilerParams(
        use_tc_tiling_on_sc=False, needs_layout_passes=False
    ),
)


@jax.jit
def kernel(sample, entity_embedding, relation_embedding):
    samp = sample.reshape(NW, BW * 3)
    score = _sc_call(samp, entity_embedding, relation_embedding)
    return score.reshape(BATCH_N, 1)


# speedup vs baseline: 1.6095x; 1.6095x over previous
import functools

import jax
import jax.numpy as jnp
from jax import lax
from jax.experimental import pallas as pl
from jax.experimental.pallas import tpu as pltpu
from jax.experimental.pallas import tpu_sc as plsc

HIDDEN = 64
GAMMA_VAL = 12.0
BATCH_N = 16384
LANES = 16

NUM_CORES = 2
NUM_SUBCORES = 16
NW = NUM_CORES * NUM_SUBCORES
BW = BATCH_N // NW
CH = 128
NCH = BW // CH
CGROUPS = CH // LANES


def _score_body(samp, ent, rel, out, samp_v, h_v, r_v, t_v, out_v, sem):
    wid = lax.axis_index("s") * NUM_CORES + lax.axis_index("c")
    base = wid * BW

    pltpu.sync_copy(samp.at[pl.ds(base * 3, BW * 3)], samp_v)

    iota = lax.iota(jnp.int32, LANES)
    stride3 = iota * 3

    def chunk(c, carry):
        s0 = c * CH

        def enqueue(g, carry):
            flat = (s0 + g * LANES) * 3
            hidx = plsc.load_gather(samp_v, [stride3 + flat])
            ridx = plsc.load_gather(samp_v, [stride3 + (flat + 1)])
            tidx = plsc.load_gather(samp_v, [stride3 + (flat + 2)])
            for j in range(LANES):
                dst = pl.ds(g * LANES + j, 1)
                pltpu.async_copy(ent.at[pl.ds(hidx[j], 1)], h_v.at[dst], sem)
                pltpu.async_copy(rel.at[pl.ds(ridx[j], 1)], r_v.at[dst], sem)
                pltpu.async_copy(ent.at[pl.ds(tidx[j], 1)], t_v.at[dst], sem)
            return carry

        lax.fori_loop(0, CGROUPS, enqueue, 0)

        def drain(i, carry):
            pltpu.make_async_copy(
                ent.at[pl.ds(0, 1)], h_v.at[pl.ds(0, 1)], sem
            ).wait()
            return carry

        lax.fori_loop(0, 3 * CH, drain, 0)

        def body(g, carry):
            res = jnp.zeros((LANES,), jnp.float32)
            for j in range(LANES):
                n = g * LANES + j
                p = None
                for k in range(HIDDEN // LANES):
                    sl = pl.ds(k * LANES, LANES)
                    v = jnp.abs(h_v[n, sl] + r_v[n, sl] - t_v[n, sl])
                    p = v if p is None else p + v
                tot = plsc.cumsum(p)[LANES - 1]
                res = jnp.where(iota == j, tot, res)
            out_v[pl.ds(s0 + g * LANES, LANES)] = GAMMA_VAL - res
            return carry

        lax.fori_loop(0, CGROUPS, body, 0)
        return carry

    lax.fori_loop(0, NCH, chunk, 0)
    pltpu.sync_copy(out_v, out.at[pl.ds(base, BW)])


_sc_call = pl.kernel(
    _score_body,
    out_type=jax.ShapeDtypeStruct((BATCH_N,), jnp.float32),
    mesh=plsc.VectorSubcoreMesh(core_axis_name="c", subcore_axis_name="s"),
    scratch_types=[
        pltpu.VMEM((BW * 3,), jnp.int32),
        pltpu.VMEM((CH, HIDDEN), jnp.float32),
        pltpu.VMEM((CH, HIDDEN), jnp.float32),
        pltpu.VMEM((CH, HIDDEN), jnp.float32),
        pltpu.VMEM((BW,), jnp.float32),
        pltpu.SemaphoreType.DMA,
    ],
    compiler_params=pltpu.CompilerParams(needs_layout_passes=False),
)


@jax.jit
def kernel(sample, entity_embedding, relation_embedding):
    samp = sample.reshape(BATCH_N * 3)
    score = _sc_call(samp, entity_embedding, relation_embedding)
    return score.reshape(BATCH_N, 1)
